# baseline (speedup 1.0000x reference)
import numpy as np
import jax
import jax.numpy as jnp
from jax import lax
from jax.experimental import pallas as pl

T = 1.3
QHAT = 0.92
LAMDA = 0.15
KREG = 5
TOPK = 12
NLEV = 4
ROWS = 8
LW = 128
UN = 8

INV_T = np.float32(1.0 / T)

_MSK = np.zeros(TOPK, np.float32)
_MSK[KREG:] = np.float32(LAMDA)
_REGCS = np.cumsum(_MSK).astype(np.float32)

_NEG_INF = np.float32(-np.inf)


def _insert(x, a):
    out = []
    cur = x
    for k in range(len(a)):
        out.append(jnp.maximum(a[k], cur))
        cur = jnp.minimum(a[k], cur)
    return tuple(out)


def _body(x_ref, u_ref, mask_ref, sizes_ref):
    n = x_ref.shape[1]
    nfull = n // LW
    tail_w = n - nfull * LW
    nu = nfull // UN
    rem = nfull - nu * UN

    if tail_w:
        tail = x_ref[:, nfull * LW:n]
        pad = jnp.full((ROWS, LW - tail_w), _NEG_INF, jnp.float32)
        a0 = jnp.concatenate([tail, pad], axis=1)
    else:
        a0 = jnp.full((ROWS, LW), _NEG_INF, jnp.float32)
    neg = jnp.full((ROWS, LW), _NEG_INF, jnp.float32)
    a_init = (a0,) + (neg,) * (NLEV - 1)

    def p1(i, a):
        for j in range(UN):
            a = _insert(x_ref[:, pl.ds((i * UN + j) * LW, LW)], a)
        return a

    a = lax.fori_loop(0, nu, p1, a_init)
    for c in range(nu * UN, nfull):
        a = _insert(x_ref[:, pl.ds(c * LW, LW)], a)
    a = list(a)

    lane = lax.broadcasted_iota(jnp.int32, (ROWS, LW), 1)
    tops = []
    for _ in range(TOPK):
        mr = jnp.max(a[0], axis=1, keepdims=True)
        il = jnp.max(jnp.where(a[0] == mr, lane, -1), axis=1, keepdims=True)
        sel = lane == il
        for k in range(NLEV - 1):
            a[k] = jnp.where(sel, a[k + 1], a[k])
        a[NLEV - 1] = jnp.where(sel, _NEG_INF, a[NLEV - 1])
        tops.append(mr)

    m_y = tops[0] / np.float32(T)
    t12 = tops[TOPK - 1]

    one = jnp.ones((ROWS, LW), jnp.int32)
    zero = jnp.zeros((ROWS, LW), jnp.int32)

    def p2(i, carry):
        acc, cnt = carry
        for j in range(UN):
            x_c = x_ref[:, pl.ds((i * UN + j) * LW, LW)]
            acc = acc + jnp.exp(x_c * INV_T - m_y)
            cnt = cnt + jnp.where(x_c > t12, one, zero)
        return acc, cnt

    acc0 = jnp.exp(a0 * INV_T - m_y)
    cnt0 = jnp.where(a0 > t12, one, zero)
    acc, cnt = lax.fori_loop(0, nu, p2, (acc0, cnt0))
    for c in range(nu * UN, nfull):
        x_c = x_ref[:, pl.ds(c * LW, LW)]
        acc = acc + jnp.exp(x_c * INV_T - m_y)
        cnt = cnt + jnp.where(x_c > t12, one, zero)
    z = jnp.sum(acc, axis=1, keepdims=True)
    n_gt = jnp.sum(cnt, axis=1, keepdims=True)

    e_gt = jnp.zeros_like(n_gt)
    for k in range(TOPK):
        e_gt = e_gt + (tops[k] > t12).astype(jnp.int32)
    ok = jnp.all(n_gt == e_gt)

    def exact_fallback():
        iota = lax.broadcasted_iota(jnp.int32, (ROWS, n), 1)
        w = x_ref[...]
        out = []
        for _ in range(TOPK):
            mk = jnp.max(w, axis=1, keepdims=True)
            ik = jnp.max(jnp.where(w == mk, iota, -1), axis=1, keepdims=True)
            w = jnp.where(iota == ik, _NEG_INF, w)
            out.append(mk)
        return tuple(out)

    tops = list(lax.cond(ok, lambda: tuple(tops), exact_fallback))

    s = [jnp.exp(t / np.float32(T) - m_y) / z for t in tops]
    cs = [s[0]]
    for k in range(1, TOPK):
        cs.append(cs[-1] + s[k])
    ord_reg = [s[k] + (np.float32(LAMDA) if k >= KREG else np.float32(0.0))
               for k in range(TOPK)]
    cs_reg = [cs[k] + _REGCS[k] for k in range(TOPK)]

    cnt_sz = jnp.zeros_like(tops[0], dtype=jnp.int32)
    for k in range(TOPK):
        cnt_sz = cnt_sz + (cs_reg[k] <= np.float32(QHAT)).astype(jnp.int32)
    sizes_base = cnt_sz + 1

    idx = sizes_base - 1
    ord_at = jnp.zeros_like(s[0])
    cs_at = jnp.zeros_like(s[0])
    for k in range(TOPK):
        sel = idx == k
        ord_at = jnp.where(sel, ord_reg[k], ord_at)
        cs_at = jnp.where(sel, cs_reg[k], cs_at)
    v = (cs_at - np.float32(QHAT)) / ord_at

    u = u_ref[...].reshape(ROWS, 1)
    sizes = sizes_base - (u <= v).astype(jnp.int32)

    cutoff = jnp.full_like(s[0], jnp.inf)
    for k in range(TOPK):
        cutoff = jnp.where(sizes - 1 == k, tops[k], cutoff)

    n_gt_cut = jnp.zeros_like(s[0])
    for k in range(TOPK):
        n_gt_cut = n_gt_cut + (tops[k] > cutoff).astype(jnp.float32)
    q = sizes.astype(jnp.float32) - n_gt_cut

    def _strict_lt(w):
        li = lax.broadcasted_iota(jnp.int32, (w, w), 0)
        lj = lax.broadcasted_iota(jnp.int32, (w, w), 1)
        return (li < lj).astype(jnp.bfloat16)

    ltm = _strict_lt(LW)

    def _mask_chunk(x_c, carry, lt):
        gt = x_c > cutoff
        eq = x_c == cutoff
        eqb = eq.astype(jnp.bfloat16)
        pref = lax.dot_general(eqb, lt, (((1,), (0,)), ((), ())),
                               preferred_element_type=jnp.float32)
        pos = carry + pref
        mask = gt | (eq & (pos < q))
        w = x_c.shape[1]
        carry = (carry + pref[:, w - 1:w]
                 + eq[:, w - 1:w].astype(jnp.float32))
        return mask, carry

    def p3(i, carry):
        for j in range(UN):
            ds = pl.ds((i * UN + j) * LW, LW)
            mask, carry = _mask_chunk(x_ref[:, ds], carry, ltm)
            mask_ref[:, ds] = mask
        return carry

    carry = lax.fori_loop(0, nu, p3, jnp.zeros_like(s[0]))
    for c in range(nu * UN, nfull):
        ds = pl.ds(c * LW, LW)
        mask, carry = _mask_chunk(x_ref[:, ds], carry, ltm)
        mask_ref[:, ds] = mask
    if tail_w:
        mask, carry = _mask_chunk(x_ref[:, nfull * LW:n], carry,
                                  _strict_lt(tail_w))
        mask_ref[:, nfull * LW:n] = mask
    sizes_ref[...] = sizes.reshape(1, 1, ROWS)


def kernel(logits):
    b, n = logits.shape
    g = b // ROWS
    u = jax.random.uniform(jax.random.key(1), (b,), dtype=logits.dtype)
    u3 = u.reshape(g, 1, ROWS)

    mask, sizes3 = pl.pallas_call(
        _body,
        grid=(g,),
        in_specs=[
            pl.BlockSpec((ROWS, n), lambda i: (i, 0)),
            pl.BlockSpec((1, 1, ROWS), lambda i: (i, 0, 0)),
        ],
        out_specs=[
            pl.BlockSpec((ROWS, n), lambda i: (i, 0)),
            pl.BlockSpec((1, 1, ROWS), lambda i: (i, 0, 0)),
        ],
        out_shape=[
            jax.ShapeDtypeStruct((b, n), jnp.bool_),
            jax.ShapeDtypeStruct((g, 1, ROWS), jnp.int32),
        ],
    )(logits, u3)

    return (logits, sizes3.reshape(b), mask)
